# baseline (speedup 1.0000x reference)
import jax
import jax.numpy as jnp
from jax.experimental import pallas as pl

P = 2048
KNN = 32
BC = 512

_SELF = 0x7F000000


def _knnsel_kernel(h_ref, htb_ref, mt_ref):
    h = h_ref[...]
    htb = htb_ref[...]
    g = jnp.dot(h, htb, preferred_element_type=jnp.float32)
    hh = h * h
    ones = jnp.ones((h.shape[1], 1), jnp.float32)
    sqc = jnp.dot(hh, ones, preferred_element_type=jnp.float32)
    sqr = jnp.sum(htb * htb, axis=0, keepdims=True)
    d2 = jnp.maximum(sqc + sqr - 2.0 * g, 0.0)
    row = jax.lax.broadcasted_iota(jnp.int32, (P, BC), 0)
    col = jax.lax.broadcasted_iota(jnp.int32, (P, BC), 1)
    col = col + BC * pl.program_id(0)
    key = jax.lax.bitcast_convert_type(d2, jnp.int32)
    key = jnp.where(col == row, jnp.int32(_SELF), key)

    sign = jnp.int32(-2147483648)

    def body(_, c):
        s = jnp.min(key - c, axis=0, keepdims=True)
        return (s + c + 1) ^ sign

    c = jax.lax.fori_loop(0, KNN, body, jnp.full((1, BC), sign))
    tp1 = c ^ sign
    mt_ref[...] = jnp.where(key < tp1, jnp.float32(1.0), jnp.float32(0.0))


def _knnsel(h, ht):
    d = h.shape[1]
    return pl.pallas_call(
        _knnsel_kernel,
        grid=(P // BC,),
        in_specs=[
            pl.BlockSpec((P, d), lambda i: (0, 0)),
            pl.BlockSpec((d, BC), lambda i: (0, i)),
        ],
        out_specs=pl.BlockSpec((P, BC), lambda i: (0, i)),
        out_shape=jax.ShapeDtypeStruct((P, P), jnp.float32),
    )(h, ht)


def _gcn_kernel(mt_ref, h_ref, w_ref, b_ref, o_ref):
    mt = mt_ref[...]
    deg = 1.0 + jnp.sum(mt, axis=1, keepdims=True)
    dinv = jax.lax.rsqrt(deg)
    xw = jnp.dot(h_ref[...], w_ref[...],
                 preferred_element_type=jnp.float32)
    z = xw * dinv
    y = jnp.dot(mt, z, preferred_element_type=jnp.float32)
    o_ref[...] = jnp.maximum(dinv * (y + z) + b_ref[...], 0.0)


def _gcn(mt, h, w, b):
    din, dout = w.shape
    return pl.pallas_call(
        _gcn_kernel,
        in_specs=[
            pl.BlockSpec((P, P), lambda: (0, 0)),
            pl.BlockSpec((P, din), lambda: (0, 0)),
            pl.BlockSpec((din, dout), lambda: (0, 0)),
            pl.BlockSpec((1, dout), lambda: (0, 0)),
        ],
        out_specs=pl.BlockSpec((P, dout), lambda: (0, 0)),
        out_shape=jax.ShapeDtypeStruct((P, dout), jnp.float32),
    )(mt, h, w, b.reshape(1, dout))


def _head_kernel(h_ref, w4_ref, b4_ref, w5_ref, b5_ref, o_ref):
    hm = jnp.mean(h_ref[...], axis=0, keepdims=True)
    t = jnp.dot(hm, w4_ref[...], preferred_element_type=jnp.float32)
    t = jnp.maximum(t + b4_ref[...], 0.0)
    o = jnp.dot(t, w5_ref[...], preferred_element_type=jnp.float32)
    o = o + b5_ref[...]
    o = o - jnp.max(o, axis=1, keepdims=True)
    e = jnp.exp(o)
    o_ref[...] = e / jnp.sum(e, axis=1, keepdims=True)


def _head(h, w4, b4, w5, b5):
    return pl.pallas_call(
        _head_kernel,
        out_shape=jax.ShapeDtypeStruct((1, 3), jnp.float32),
    )(h, w4, b4.reshape(1, -1), w5, b5.reshape(1, -1))


def kernel(x, W1, b1, W2, b2, W3, b3, W4, b4, W5, b5):
    h = x[0]
    for w, b in ((W1, b1), (W2, b2), (W3, b3)):
        mt = _knnsel(h, h.T)
        h = _gcn(mt, h, w, b)
    return _head(h, W4, b4, W5, b5)


# speedup vs baseline: 1.0027x; 1.0027x over previous
import jax
import jax.numpy as jnp
from jax.experimental import pallas as pl

P = 2048
KNN = 32
BC = 256

_SELF = 0x7F000000


def _knnsel_kernel(h_ref, htb_ref, mt_ref):
    h = h_ref[...]
    htb = htb_ref[...]
    g = jnp.dot(h, htb, preferred_element_type=jnp.float32)
    hh = h * h
    ones = jnp.ones((h.shape[1], 1), jnp.float32)
    sqc = jnp.dot(hh, ones, preferred_element_type=jnp.float32)
    sqr = jnp.sum(htb * htb, axis=0, keepdims=True)
    d2 = jnp.maximum(sqc + sqr - 2.0 * g, 0.0)
    row = jax.lax.broadcasted_iota(jnp.int32, (P, BC), 0)
    col = jax.lax.broadcasted_iota(jnp.int32, (P, BC), 1)
    col = col + BC * pl.program_id(0)
    key = jax.lax.bitcast_convert_type(d2, jnp.int32)
    key = jnp.where(col == row, jnp.int32(_SELF), key)

    sign = jnp.int32(-2147483648)

    def body(_, c):
        s = jnp.min(key - c, axis=0, keepdims=True)
        return (s + c + 1) ^ sign

    c = jax.lax.fori_loop(0, KNN, body, jnp.full((1, BC), sign))
    tp1 = c ^ sign
    mt_ref[...] = jnp.where(key < tp1, jnp.float32(1.0), jnp.float32(0.0))


def _knnsel(h, ht):
    d = h.shape[1]
    return pl.pallas_call(
        _knnsel_kernel,
        grid=(P // BC,),
        in_specs=[
            pl.BlockSpec((P, d), lambda i: (0, 0)),
            pl.BlockSpec((d, BC), lambda i: (0, i)),
        ],
        out_specs=pl.BlockSpec((P, BC), lambda i: (0, i)),
        out_shape=jax.ShapeDtypeStruct((P, P), jnp.float32),
    )(h, ht)


def _gcn_kernel(mt_ref, h_ref, w_ref, b_ref, o_ref):
    mt = mt_ref[...]
    deg = 1.0 + jnp.sum(mt, axis=1, keepdims=True)
    dinv = jax.lax.rsqrt(deg)
    xw = jnp.dot(h_ref[...], w_ref[...],
                 preferred_element_type=jnp.float32)
    z = xw * dinv
    y = jnp.dot(mt, z, preferred_element_type=jnp.float32)
    o_ref[...] = jnp.maximum(dinv * (y + z) + b_ref[...], 0.0)


def _gcn(mt, h, w, b):
    din, dout = w.shape
    return pl.pallas_call(
        _gcn_kernel,
        in_specs=[
            pl.BlockSpec((P, P), lambda: (0, 0)),
            pl.BlockSpec((P, din), lambda: (0, 0)),
            pl.BlockSpec((din, dout), lambda: (0, 0)),
            pl.BlockSpec((1, dout), lambda: (0, 0)),
        ],
        out_specs=pl.BlockSpec((P, dout), lambda: (0, 0)),
        out_shape=jax.ShapeDtypeStruct((P, dout), jnp.float32),
    )(mt, h, w, b.reshape(1, dout))


def _head_kernel(h_ref, w4_ref, b4_ref, w5_ref, b5_ref, o_ref):
    hm = jnp.mean(h_ref[...], axis=0, keepdims=True)
    t = jnp.dot(hm, w4_ref[...], preferred_element_type=jnp.float32)
    t = jnp.maximum(t + b4_ref[...], 0.0)
    o = jnp.dot(t, w5_ref[...], preferred_element_type=jnp.float32)
    o = o + b5_ref[...]
    o = o - jnp.max(o, axis=1, keepdims=True)
    e = jnp.exp(o)
    o_ref[...] = e / jnp.sum(e, axis=1, keepdims=True)


def _head(h, w4, b4, w5, b5):
    return pl.pallas_call(
        _head_kernel,
        out_shape=jax.ShapeDtypeStruct((1, 3), jnp.float32),
    )(h, w4, b4.reshape(1, -1), w5, b5.reshape(1, -1))


def kernel(x, W1, b1, W2, b2, W3, b3, W4, b4, W5, b5):
    h = x[0]
    for w, b in ((W1, b1), (W2, b2), (W3, b3)):
        mt = _knnsel(h, h.T)
        h = _gcn(mt, h, w, b)
    return _head(h, W4, b4, W5, b5)
